# baseline (speedup 1.0000x reference)
import functools

import jax
import jax.numpy as jnp
from jax import lax
from jax.experimental import pallas as pl
from jax.experimental.pallas import tpu as pltpu
from jax.experimental.pallas import tpu_sc as plsc

_VOCAB = 1000
_D = 128
_B = 1024
_L = 50
_TOK = _B * _L
_NW = 32
_PER_W = _TOK // _NW
_C = 40
_NCH = _PER_W // _C


def _logit_table_body(emb_ref, w_ref, b_ref, out_ref):
    out_ref[...] = lax.dot_general(
        emb_ref[...], w_ref[...], (((1,), (1,)), ((), ())),
        preferred_element_type=jnp.float32) + b_ref[...]


def _logit_table(emb, w, b2d):
    return pl.pallas_call(
        _logit_table_body,
        out_shape=jax.ShapeDtypeStruct((_VOCAB, _VOCAB), jnp.float32),
    )(emb, w, b2d)


def _make_gather():
    mesh = plsc.VectorSubcoreMesh(core_axis_name="c", subcore_axis_name="s")

    @functools.partial(
        pl.kernel,
        out_type=jax.ShapeDtypeStruct((_TOK, _VOCAB), jnp.float32),
        mesh=mesh,
        compiler_params=pltpu.CompilerParams(use_tc_tiling_on_sc=False),
        scratch_types=[
            pltpu.VMEM((_NCH, _C), jnp.int32),
            pltpu.VMEM((_C, _VOCAB), jnp.float32),
            pltpu.VMEM((_C, _VOCAB), jnp.float32),
            pltpu.SemaphoreType.DMA,
            pltpu.SemaphoreType.DMA,
        ],
    )
    def gather(ids_hbm, table_hbm, out_hbm, idx_v, buf0, buf1, sem0, sem1):
        wid = lax.axis_index("s") * 2 + lax.axis_index("c")
        base = wid * _PER_W
        pltpu.sync_copy(ids_hbm.at[wid], idx_v)

        def g(c, buf, sem):
            return pltpu.make_async_copy(table_hbm.at[idx_v.at[c]], buf, sem)

        def w(c, buf):
            pltpu.sync_copy(buf, out_hbm.at[pl.ds(base + c * _C, _C)])

        g(0, buf0, sem0).start()

        def body(p, carry):
            a = 2 * p
            g(a + 1, buf1, sem1).start()
            g(a, buf0, sem0).wait()
            w(a, buf0)
            g(a + 2, buf0, sem0).start()
            g(a + 1, buf1, sem1).wait()
            w(a + 1, buf1)
            return carry

        lax.fori_loop(0, _NCH // 2 - 1, body, 0)
        last = _NCH - 1
        g(last, buf1, sem1).start()
        g(last - 1, buf0, sem0).wait()
        w(last - 1, buf0)
        g(last, buf1, sem1).wait()
        w(last, buf1)

    return gather


_gather = _make_gather()


def kernel(input_ids, emb_table, head_w, head_b):
    ids = input_ids.astype(jnp.int32).reshape(_NW, _NCH, _C)
    table = _logit_table(emb_table, head_w, head_b.reshape(1, _VOCAB))
    out = _gather(ids, table)
    return out.reshape(_B, _L, _VOCAB)


# speedup vs baseline: 1.8129x; 1.8129x over previous
import functools

import jax
import jax.numpy as jnp
from jax import lax
from jax.experimental import pallas as pl
from jax.experimental.pallas import tpu as pltpu
from jax.experimental.pallas import tpu_sc as plsc

_VOCAB = 1000
_D = 128
_B = 1024
_L = 50
_TOK = _B * _L
_NW = 32
_PER_W = _TOK // _NW
_C = 64
_NCH = _PER_W // _C

_BB = 16
_TB = _BB * _L


def _make_gather():
    mesh = plsc.VectorSubcoreMesh(core_axis_name="c", subcore_axis_name="s")

    @functools.partial(
        pl.kernel,
        out_type=jax.ShapeDtypeStruct((_TOK, _D), jnp.float32),
        mesh=mesh,
        scratch_types=[
            pltpu.VMEM((_NCH, _C), jnp.int32),
            pltpu.VMEM((_C, _D), jnp.float32),
            pltpu.VMEM((_C, _D), jnp.float32),
            pltpu.SemaphoreType.DMA,
            pltpu.SemaphoreType.DMA,
        ],
    )
    def gather(ids_hbm, emb_hbm, x_hbm, idx_v, buf0, buf1, sem0, sem1):
        wid = lax.axis_index("s") * 2 + lax.axis_index("c")
        base = wid * _PER_W
        pltpu.sync_copy(ids_hbm.at[wid], idx_v)

        def g(c, buf, sem):
            return pltpu.make_async_copy(emb_hbm.at[idx_v.at[c]], buf, sem)

        def w(c, buf):
            pltpu.sync_copy(buf, x_hbm.at[pl.ds(base + c * _C, _C)])

        g(0, buf0, sem0).start()

        def body(p, carry):
            a = 2 * p
            g(a + 1, buf1, sem1).start()
            g(a, buf0, sem0).wait()
            w(a, buf0)
            g(a + 2, buf0, sem0).start()
            g(a + 1, buf1, sem1).wait()
            w(a + 1, buf1)
            return carry

        lax.fori_loop(0, (_NCH - 3) // 2, body, 0)
        g(_NCH - 2, buf1, sem1).start()
        g(_NCH - 3, buf0, sem0).wait()
        w(_NCH - 3, buf0)
        g(_NCH - 1, buf0, sem0).start()
        g(_NCH - 2, buf1, sem1).wait()
        w(_NCH - 2, buf1)
        g(_NCH - 1, buf0, sem0).wait()
        w(_NCH - 1, buf0)

    return gather


_gather = _make_gather()


def _head_body(x_ref, w_ref, b_ref, out_ref):
    res = lax.dot_general(
        x_ref[...], w_ref[...], (((1,), (1,)), ((), ())),
        preferred_element_type=jnp.float32)
    out_ref[...] = res.reshape(_BB, _L, _VOCAB) + b_ref[...]


def _head(x, w, b2d):
    return pl.pallas_call(
        _head_body,
        grid=(_B // _BB,),
        in_specs=[
            pl.BlockSpec((_TB, _D), lambda i: (i, 0)),
            pl.BlockSpec((_VOCAB, _D), lambda i: (0, 0)),
            pl.BlockSpec((1, _VOCAB), lambda i: (0, 0)),
        ],
        out_specs=pl.BlockSpec((_BB, _L, _VOCAB), lambda i: (i, 0, 0)),
        out_shape=jax.ShapeDtypeStruct((_B, _L, _VOCAB), jnp.float32),
    )(x, w, b2d)


def kernel(input_ids, emb_table, head_w, head_b):
    ids = input_ids.astype(jnp.int32).reshape(_NW, _NCH, _C)
    x = _gather(ids, emb_table)
    return _head(x, head_w, head_b.reshape(1, _VOCAB))
